# baseline (speedup 1.0000x reference)
import functools

import jax
import jax.numpy as jnp
from jax import lax
from jax.experimental import pallas as pl
from jax.experimental.pallas import tpu as pltpu
from jax.experimental.pallas import tpu_sc as plsc

_FIELD = 100000
_NF = 26
_D = 16
_ROWS = 16384
_B = _ROWS * _NF


@functools.cache
def _build():
    info = plsc.get_sparse_core_info()
    nc, ns = info.num_cores, info.num_subcores
    nw = nc * ns
    bpw = _B // nw
    nch = 8
    chunk = bpw // nch

    mesh = plsc.VectorSubcoreMesh(core_axis_name="c", subcore_axis_name="s")

    @functools.partial(
        pl.kernel,
        mesh=mesh,
        compiler_params=pltpu.CompilerParams(use_tc_tiling_on_sc=False),
        out_type=jax.ShapeDtypeStruct((_B, _D), jnp.float32),
        scratch_types=[
            pltpu.VMEM((chunk,), jnp.int32),
            pltpu.VMEM((chunk,), jnp.int32),
            pltpu.VMEM((chunk,), jnp.int32),
            pltpu.VMEM((chunk, _D), jnp.float32),
            pltpu.SemaphoreType.DMA,
        ],
    )
    def k(x_hbm, off_hbm, table_hbm, out_hbm, xv, offv, idxv, rowsv, sem):
        wid = lax.axis_index("s") * nc + lax.axis_index("c")
        wbase = wid * bpw
        pltpu.sync_copy(off_hbm, offv)

        def do_chunk(c, carry):
            base = wbase + c * chunk
            pltpu.sync_copy(x_hbm.at[pl.ds(base, chunk)], xv)

            def add(i, carry2):
                s = pl.ds(i * 16, 16)
                idxv[s] = xv[s] + offv[s]
                return carry2

            lax.fori_loop(0, chunk // 16, add, 0)
            pltpu.async_copy(table_hbm.at[idxv], rowsv, sem).wait()
            pltpu.sync_copy(rowsv, out_hbm.at[pl.ds(base, chunk)])
            return carry

        lax.fori_loop(0, nch, do_chunk, 0)

    off_tile = jnp.tile(jnp.arange(_NF, dtype=jnp.int32) * _FIELD, chunk // _NF)
    return k, off_tile


def kernel(x, table):
    k, off_tile = _build()
    out = k(x.reshape(_B), off_tile, table)
    return out.reshape(_ROWS, _NF, _D)


# speedup vs baseline: 1.2610x; 1.2610x over previous
import functools

import jax
import jax.numpy as jnp
from jax import lax
from jax.experimental import pallas as pl
from jax.experimental.pallas import tpu as pltpu
from jax.experimental.pallas import tpu_sc as plsc

_FIELD = 100000
_NF = 26
_D = 16
_ROWS = 16384
_BPW = 512
_NBT = _BPW // 128


@functools.cache
def _build():
    info = plsc.get_sparse_core_info()
    nc, ns = info.num_cores, info.num_subcores
    nw = nc * ns
    assert _ROWS == nw * _BPW

    mesh = plsc.VectorSubcoreMesh(core_axis_name="c", subcore_axis_name="s")

    @functools.partial(
        pl.kernel,
        mesh=mesh,
        compiler_params=pltpu.CompilerParams(
            use_tc_tiling_on_sc=False, needs_layout_passes=False),
        out_type=jax.ShapeDtypeStruct((_NF, 2, 128, 8, 128), jnp.float32),
        scratch_types=[
            pltpu.VMEM((_NF, _BPW), jnp.int32),
            pltpu.VMEM((2, _BPW), jnp.int32),
            pltpu.VMEM((2, _BPW, _D), jnp.float32),
            pltpu.VMEM((2, 2, _NBT, 8, 128), jnp.float32),
            pltpu.SemaphoreType.DMA,
            pltpu.SemaphoreType.DMA,
        ],
    )
    def k(xt_hbm, table_hbm, out_hbm, xv, idxv, rowsv, outv, gsem, osem):
        wid = lax.axis_index("s") * nc + lax.axis_index("c")
        b0 = wid * _BPW
        bt0 = wid * _NBT
        pltpu.sync_copy(xt_hbm.at[:, pl.ds(b0, _BPW)], xv)

        def do_field(f, carry):
            buf = lax.rem(f, 2)

            def add(g, carry2):
                s = pl.ds(g * 16, 16)
                idxv[buf, s] = xv[f, s] + f * _FIELD
                return carry2

            lax.fori_loop(0, _BPW // 16, add, 0)
            pltpu.async_copy(
                table_hbm.at[idxv.at[buf]], rowsv.at[buf], gsem).wait()

            lane = lax.iota(jnp.int32, 16)

            def trans(t, carry3):
                bgrp = lax.rem(t, 8)
                ch = lax.rem(t // 8, 8)
                bt = lax.rem(t // 64, _NBT)
                cg = t // (64 * _NBT)
                ridx = lane + (bt * 128 + bgrp * 16)
                cidx = jnp.full((16,), cg * 8 + ch, jnp.int32)
                outv[buf, cg, bt, ch, pl.ds(bgrp * 16, 16)] = plsc.load_gather(
                    rowsv.at[buf], [ridx, cidx])
                return carry3

            lax.fori_loop(0, 2 * _NBT * 8 * 8, trans, 0)

            pltpu.async_copy(
                outv.at[buf],
                out_hbm.at[f, :, pl.ds(bt0, _NBT)], osem)

            @pl.when(f > 0)
            def _():
                pltpu.make_async_copy(
                    outv.at[lax.rem(f + 1, 2)],
                    out_hbm.at[0, :, pl.ds(bt0, _NBT)], osem).wait()

            return carry

        lax.fori_loop(0, _NF, do_field, 0)
        pltpu.make_async_copy(
            outv.at[lax.rem(_NF - 1, 2)],
            out_hbm.at[0, :, pl.ds(bt0, _NBT)], osem).wait()

    return k


def kernel(x, table):
    k = _build()
    out5 = k(x.T, table)
    return out5.transpose(2, 4, 0, 1, 3).reshape(_ROWS, _NF, _D)
